# baseline (speedup 1.0000x reference)
import functools

import jax
import jax.numpy as jnp
from jax import lax
from jax.experimental import pallas as pl
from jax.experimental.pallas import tpu as pltpu
from jax.experimental.pallas import tpu_sc as plsc

_F32 = jnp.float32
_I32 = jnp.int32

_NTILES = 16
_K = 64
_HQ = 64


def _feat1_body(x_ref, w_ref, av_ref, hlo_ref, hhi_ref, sd_ref):
    h = jnp.dot(x_ref[...], w_ref[...], preferred_element_type=_F32)
    hlo_ref[...] = h[:, :2 * _HQ]
    hhi_ref[...] = h[:, 2 * _HQ:]
    sd_ref[...] = jnp.dot(h, av_ref[...], preferred_element_type=_F32)


def _feat1(x, w, av):
    n = x.shape[0]
    return pl.pallas_call(
        _feat1_body,
        out_shape=[jax.ShapeDtypeStruct((n, 2 * _HQ), _F32)] * 2
        + [jax.ShapeDtypeStruct((n, 2), _F32)],
    )(x, w, av)


def _feat2_body(a0_ref, a1_ref, a2_ref, a3_ref, den_ref, b_ref,
                w0_ref, w1_ref, w2_ref, w3_ref, av_ref,
                hlo_ref, hhi_ref, sd_ref):
    den = 0.5 * jnp.sum(den_ref[...], axis=1, keepdims=True)
    inv = 1.0 / (den + 1e-16)
    b = b_ref[...]
    acc = None
    for q, (a_ref, w_ref) in enumerate(
            [(a0_ref, w0_ref), (a1_ref, w1_ref),
             (a2_ref, w2_ref), (a3_ref, w3_ref)]):
        hq = jnp.maximum(a_ref[...] * inv + b[:, q * _HQ:(q + 1) * _HQ], 0.0)
        part = jnp.dot(hq, w_ref[...], preferred_element_type=_F32)
        acc = part if acc is None else acc + part
    h = acc
    hlo_ref[...] = h[:, :2 * _HQ]
    hhi_ref[...] = h[:, 2 * _HQ:]
    sd_ref[...] = jnp.dot(h, av_ref[...], preferred_element_type=_F32)


def _feat2(a0, a1, a2, a3, den, b, w0, w1, w2, w3, av):
    n = a0.shape[0]
    return pl.pallas_call(
        _feat2_body,
        out_shape=[jax.ShapeDtypeStruct((n, 2 * _HQ), _F32)] * 2
        + [jax.ShapeDtypeStruct((n, 2), _F32)],
    )(a0, a1, a2, a3, den, b, w0, w1, w2, w3, av)


def _tail_body(a0_ref, a1_ref, a2_ref, a3_ref, den_ref, b_ref, batch_ref,
               z_ref, wz1_ref, bz1_ref, wz2_ref, bz2_ref,
               wx0_ref, wx1_ref, wx2q_ref, wx3q_ref, wxb_ref, bxz1_ref,
               wxz2_ref, bxz2_ref, wxz3_ref, bxz3_ref, out_ref):
    den = 0.5 * jnp.sum(den_ref[...], axis=1, keepdims=True)
    inv = 1.0 / (den + 1e-16)
    b = b_ref[...]
    g = 64
    n = a0_ref.shape[0]
    oneh = (lax.broadcasted_iota(_I32, (g, n), 0)
            == batch_ref[...]).astype(_F32)
    cnt = jnp.sum(oneh, axis=1, keepdims=True)
    scale = 1.0 / jnp.maximum(cnt, 1.0)
    acc = None
    for q, (a_ref, wx_ref) in enumerate(
            [(a0_ref, wx0_ref), (a1_ref, wx1_ref),
             (a2_ref, wx2q_ref), (a3_ref, wx3q_ref)]):
        hq = jnp.maximum(a_ref[...] * inv + b[:, q * _HQ:(q + 1) * _HQ], 0.0)
        xgq = jnp.dot(oneh, hq, preferred_element_type=_F32) * scale
        part = jnp.dot(xgq, wx_ref[...], preferred_element_type=_F32)
        acc = part if acc is None else acc + part
    zz = jnp.maximum(jnp.dot(z_ref[...], wz1_ref[...],
                             preferred_element_type=_F32) + bz1_ref[...], 0.0)
    zz = jnp.maximum(jnp.dot(zz, wz2_ref[...],
                             preferred_element_type=_F32) + bz2_ref[...], 0.0)
    t1 = (acc + jnp.dot(zz, wxb_ref[...], preferred_element_type=_F32)
          + bxz1_ref[...])
    t1 = jnp.maximum(t1, 0.0)
    t2 = jnp.maximum(jnp.dot(t1, wxz2_ref[...],
                             preferred_element_type=_F32) + bxz2_ref[...], 0.0)
    out_ref[...] = (jnp.dot(t2, wxz3_ref[...], preferred_element_type=_F32)
                    + bxz3_ref[...])


def _tail(a0, a1, a2, a3, den, b, batch2d, z, wz1, bz1, wz2, bz2,
          wx0, wx1, wx2q, wx3q, wxb, bxz1, wxz2, bxz2, wxz3, bxz3):
    g = z.shape[0]
    return pl.pallas_call(
        _tail_body,
        out_shape=jax.ShapeDtypeStruct((g, 1), _F32),
    )(a0, a1, a2, a3, den, b, batch2d, z, wz1, bz1, wz2, bz2,
      wx0, wx1, wx2q, wx3q, wxb, bxz1, wxz2, bxz2, wxz3, bxz3)


def _edge_sc(sd3, asvec, advec, hlo, hhi, zvec, zmat, ea):
    n = asvec.shape[0]
    nbrows = sd3.shape[1]
    nb = 2 * nbrows
    k = _K
    half = n // 2
    bpt = half // _NTILES - (half // _NTILES) % 8
    last = half - bpt * (_NTILES - 1) - bpt
    dr = zvec.shape[0]
    mesh = plsc.VectorSubcoreMesh(core_axis_name="c", subcore_axis_name="s")

    @functools.partial(
        pl.kernel,
        out_type=[
            jax.ShapeDtypeStruct((4, half, 128), _F32),
            jax.ShapeDtypeStruct((_NTILES, dr, 128), _F32),
        ],
        mesh=mesh,
        compiler_params=pltpu.CompilerParams(needs_layout_passes=False),
        scratch_types=[
            pltpu.VMEM((nbrows, 128), _I32),
            pltpu.VMEM((n,), _F32),
            pltpu.VMEM((n,), _F32),
            pltpu.VMEM((dr, 128), _F32),
            [pltpu.VMEM((k,), _I32)] * 2,
            [pltpu.VMEM((k,), _I32)] * 2,
            [pltpu.VMEM((k,), _F32)] * 2,
            [pltpu.VMEM((k,), _F32)] * 2,
            [pltpu.VMEM((k, 2 * _HQ), _F32)] * 2,
            [pltpu.VMEM((k, 2 * _HQ), _F32)] * 2,
            pltpu.VMEM_SHARED((half, 128), _F32),
            [pltpu.SemaphoreType.DMA] * 2,
            [pltpu.SemaphoreType.DMA] * 2,
        ],
    )
    def ker(sd_r, as_r, ad_r, hlo_r, hhi_r, zv_r, zm_r,
            out_r, denp_r,
            sdv, asv, adv, denv, sidx, didx, eebuf, parbuf, rowb, scbuf,
            aggsh, sem1, sem2):
        c = lax.axis_index("c")
        s = lax.axis_index("s")

        pltpu.sync_copy(sd_r.at[s], sdv)
        pltpu.sync_copy(as_r, asv)
        pltpu.sync_copy(ad_r, adv)
        pltpu.sync_copy(zv_r, denv)

        zf = jnp.zeros((16,), _F32)

        @pl.loop(0, 2)
        def _phase(phase):
            off = phase * _HQ
            pltpu.sync_copy(zm_r.at[pl.ds(s * bpt, bpt)],
                            aggsh.at[pl.ds(s * bpt, bpt)])

            @pl.when(s == _NTILES - 1)
            def _():
                pltpu.sync_copy(zm_r.at[pl.ds(_NTILES * bpt, last)],
                                aggsh.at[pl.ds(_NTILES * bpt, last)])

            plsc.subcore_barrier()

            def _prep(b, i):
                for j in range(k // 16):
                    sd = sdv[b >> 1, pl.ds((b & 1) * k + j * 16, 16)]
                    sv = lax.shift_right_logical(sd, 16)
                    dv = lax.bitwise_and(sd, 0xFFFF)
                    sidx[i][pl.ds(j * 16, 16)] = sv
                    didx[i][pl.ds(j * 16, 16)] = (
                        lax.shift_right_logical(dv, 1))
                    parbuf[i][pl.ds(j * 16, 16)] = lax.convert_element_type(
                        lax.bitwise_and(dv, 1), _F32)
                    u = (plsc.load_gather(asv, [sv])
                         + plsc.load_gather(adv, [dv]))
                    e = jnp.where(u > 0, u, 0.2 * u)
                    ee = jnp.exp(e)
                    gid = (lax.broadcast((s * nb + b) * k + j * 16, (16,))
                           + lax.broadcasted_iota(_I32, (16,), 0))
                    ee = jnp.where(gid < ea, ee, 0.0)
                    eebuf[i][pl.ds(j * 16, 16)] = ee
                    plsc.addupdate_scatter(
                        denv,
                        [lax.shift_right_logical(dv, 7),
                         lax.bitwise_and(dv, 127)], ee)

            def _gather_start(i):
                @pl.when(c == 0)
                def _():
                    pltpu.async_copy(hlo_r.at[sidx[i]], rowb[i], sem1[i])

                @pl.when(c == 1)
                def _():
                    pltpu.async_copy(hhi_r.at[sidx[i]], rowb[i], sem1[i])

            def _scale_all(i):
                for row in range(k):
                    ridx = jnp.full((16,), row, _I32)
                    spl = plsc.load_gather(eebuf[i], [ridx])
                    pv = plsc.load_gather(parbuf[i], [ridx])
                    meven = pv < 0.5
                    for u2 in range(_HQ // 16):
                        sc = rowb[i][row, pl.ds(off + u2 * 16, 16)] * spl
                        lo = jnp.where(meven, sc, zf)
                        scbuf[i][row, pl.ds(u2 * 16, 16)] = lo
                        scbuf[i][row, pl.ds(_HQ + u2 * 16, 16)] = sc - lo

            @pl.loop(0, nbrows)
            def _pair(p):
                b0 = p * 2
                _prep(b0, 0)
                _gather_start(0)
                _prep(b0 + 1, 1)
                _gather_start(1)
                pltpu.make_async_copy(hlo_r.at[sidx[0]], rowb[0],
                                      sem1[0]).wait()
                _scale_all(0)
                pltpu.async_copy(scbuf[0], aggsh.at[didx[0]], sem2[0],
                                 add=True)
                pltpu.make_async_copy(hlo_r.at[sidx[1]], rowb[1],
                                      sem1[1]).wait()
                _scale_all(1)
                pltpu.async_copy(scbuf[1], aggsh.at[didx[1]], sem2[1],
                                 add=True)
                pltpu.make_async_copy(scbuf[0], aggsh.at[didx[0]],
                                      sem2[0]).wait()
                pltpu.make_async_copy(scbuf[1], aggsh.at[didx[1]],
                                      sem2[1]).wait()

            plsc.subcore_barrier()
            q = c * 2 + phase
            pltpu.sync_copy(aggsh.at[pl.ds(s * bpt, bpt)],
                            out_r.at[q, pl.ds(s * bpt, bpt)])

            @pl.when(s == _NTILES - 1)
            def _():
                pltpu.sync_copy(aggsh.at[pl.ds(_NTILES * bpt, last)],
                                out_r.at[q, pl.ds(_NTILES * bpt, last)])

        @pl.when(c == 0)
        def _():
            pltpu.sync_copy(denv, denp_r.at[s])

    return ker(sd3, asvec, advec, hlo, hhi, zvec, zmat)


def kernel(x, z, edge_index, batch, W1, a_src1, a_dst1, b1,
           W2, a_src2, a_dst2, b2, Wz1, bz1, Wz2, bz2,
           Wxz1, bxz1, Wxz2, bxz2, Wxz3, bxz3):
    n = x.shape[0]
    hdim = W1.shape[1]
    e = edge_index.shape[1]
    ea = e + n
    chunk = _NTILES * _K
    nb = (ea + chunk - 1) // chunk
    nb = nb + (nb % 2)
    ep = nb * chunk

    loop = jnp.arange(n, dtype=edge_index.dtype)
    src = jnp.concatenate(
        [edge_index[0], loop, jnp.zeros((ep - ea,), edge_index.dtype)])
    dst = jnp.concatenate(
        [edge_index[1], loop, jnp.zeros((ep - ea,), edge_index.dtype)])
    sd3 = ((src.astype(_I32) << 16) | dst.astype(_I32)).reshape(
        _NTILES, nb // 2, 2 * _K)

    av1 = jnp.stack([a_src1, a_dst1], axis=1)
    av2 = jnp.stack([a_src2, a_dst2], axis=1)
    dr = (n + 127) // 128
    zvec = jnp.zeros((dr, 128), _F32)
    zmat = jnp.zeros((n // 2, 128), _F32)

    def _den_t(denp):
        return denp.reshape(_NTILES, dr * 128).T[:n]

    def _quarters(out4):
        return [out4[q].reshape(n, _HQ) for q in range(4)]

    hlo1, hhi1, sd1 = _feat1(x, W1, av1)
    out4_1, denp1 = _edge_sc(sd3, sd1[:, 0], sd1[:, 1],
                             hlo1, hhi1, zvec, zmat, ea)
    a1q = _quarters(out4_1)

    hlo2, hhi2, sd2 = _feat2(
        a1q[0], a1q[1], a1q[2], a1q[3], _den_t(denp1), b1[None, :],
        W2[0 * _HQ:1 * _HQ], W2[1 * _HQ:2 * _HQ],
        W2[2 * _HQ:3 * _HQ], W2[3 * _HQ:4 * _HQ], av2)
    out4_2, denp2 = _edge_sc(sd3, sd2[:, 0], sd2[:, 1],
                             hlo2, hhi2, zvec, zmat, ea)
    a2q = _quarters(out4_2)

    pred = _tail(a2q[0], a2q[1], a2q[2], a2q[3], _den_t(denp2), b2[None, :],
                 batch[None, :], z, Wz1, bz1[None, :], Wz2, bz2[None, :],
                 Wxz1[0 * _HQ:1 * _HQ], Wxz1[1 * _HQ:2 * _HQ],
                 Wxz1[2 * _HQ:3 * _HQ], Wxz1[3 * _HQ:4 * _HQ],
                 Wxz1[hdim:], bxz1[None, :],
                 Wxz2, bxz2[None, :], Wxz3, bxz3[None, :])
    return pred


# speedup vs baseline: 1.4365x; 1.4365x over previous
import functools

import jax
import jax.numpy as jnp
from jax import lax
from jax.experimental import pallas as pl
from jax.experimental.pallas import tpu as pltpu
from jax.experimental.pallas import tpu_sc as plsc

_F32 = jnp.float32
_I32 = jnp.int32

_NTILES = 16
_K = 64
_HQ = 64


def _feat1_body(x_ref, w_ref, av_ref, hlo_ref, hhi_ref, sd_ref):
    h = jnp.dot(x_ref[...], w_ref[...], preferred_element_type=_F32)
    hlo_ref[...] = h[:, :2 * _HQ]
    hhi_ref[...] = h[:, 2 * _HQ:]
    sd_ref[...] = jnp.dot(h, av_ref[...], preferred_element_type=_F32)


def _feat1(x, w, av):
    n = x.shape[0]
    return pl.pallas_call(
        _feat1_body,
        out_shape=[jax.ShapeDtypeStruct((n, 2 * _HQ), _F32)] * 2
        + [jax.ShapeDtypeStruct((n, 2), _F32)],
    )(x, w, av)


def _feat2_body(a0_ref, a1_ref, a2_ref, a3_ref, den_ref, b_ref,
                w0_ref, w1_ref, w2_ref, w3_ref, av_ref,
                hlo_ref, hhi_ref, sd_ref):
    den = 0.5 * jnp.sum(den_ref[...], axis=1, keepdims=True)
    inv = 1.0 / (den + 1e-16)
    b = b_ref[...]
    acc = None
    for q, (a_ref, w_ref) in enumerate(
            [(a0_ref, w0_ref), (a1_ref, w1_ref),
             (a2_ref, w2_ref), (a3_ref, w3_ref)]):
        hq = jnp.maximum(a_ref[...] * inv + b[:, q * _HQ:(q + 1) * _HQ], 0.0)
        part = jnp.dot(hq, w_ref[...], preferred_element_type=_F32)
        acc = part if acc is None else acc + part
    h = acc
    hlo_ref[...] = h[:, :2 * _HQ]
    hhi_ref[...] = h[:, 2 * _HQ:]
    sd_ref[...] = jnp.dot(h, av_ref[...], preferred_element_type=_F32)


def _feat2(a0, a1, a2, a3, den, b, w0, w1, w2, w3, av):
    n = a0.shape[0]
    return pl.pallas_call(
        _feat2_body,
        out_shape=[jax.ShapeDtypeStruct((n, 2 * _HQ), _F32)] * 2
        + [jax.ShapeDtypeStruct((n, 2), _F32)],
    )(a0, a1, a2, a3, den, b, w0, w1, w2, w3, av)


def _tail_body(a0_ref, a1_ref, a2_ref, a3_ref, den_ref, b_ref, batch_ref,
               z_ref, wz1_ref, bz1_ref, wz2_ref, bz2_ref,
               wx0_ref, wx1_ref, wx2q_ref, wx3q_ref, wxb_ref, bxz1_ref,
               wxz2_ref, bxz2_ref, wxz3_ref, bxz3_ref, out_ref):
    den = 0.5 * jnp.sum(den_ref[...], axis=1, keepdims=True)
    inv = 1.0 / (den + 1e-16)
    b = b_ref[...]
    g = 64
    n = a0_ref.shape[0]
    oneh = (lax.broadcasted_iota(_I32, (g, n), 0)
            == batch_ref[...]).astype(_F32)
    cnt = jnp.sum(oneh, axis=1, keepdims=True)
    scale = 1.0 / jnp.maximum(cnt, 1.0)
    acc = None
    for q, (a_ref, wx_ref) in enumerate(
            [(a0_ref, wx0_ref), (a1_ref, wx1_ref),
             (a2_ref, wx2q_ref), (a3_ref, wx3q_ref)]):
        hq = jnp.maximum(a_ref[...] * inv + b[:, q * _HQ:(q + 1) * _HQ], 0.0)
        xgq = jnp.dot(oneh, hq, preferred_element_type=_F32) * scale
        part = jnp.dot(xgq, wx_ref[...], preferred_element_type=_F32)
        acc = part if acc is None else acc + part
    zz = jnp.maximum(jnp.dot(z_ref[...], wz1_ref[...],
                             preferred_element_type=_F32) + bz1_ref[...], 0.0)
    zz = jnp.maximum(jnp.dot(zz, wz2_ref[...],
                             preferred_element_type=_F32) + bz2_ref[...], 0.0)
    t1 = (acc + jnp.dot(zz, wxb_ref[...], preferred_element_type=_F32)
          + bxz1_ref[...])
    t1 = jnp.maximum(t1, 0.0)
    t2 = jnp.maximum(jnp.dot(t1, wxz2_ref[...],
                             preferred_element_type=_F32) + bxz2_ref[...], 0.0)
    out_ref[...] = (jnp.dot(t2, wxz3_ref[...], preferred_element_type=_F32)
                    + bxz3_ref[...])


def _tail(a0, a1, a2, a3, den, b, batch2d, z, wz1, bz1, wz2, bz2,
          wx0, wx1, wx2q, wx3q, wxb, bxz1, wxz2, bxz2, wxz3, bxz3):
    g = z.shape[0]
    return pl.pallas_call(
        _tail_body,
        out_shape=jax.ShapeDtypeStruct((g, 1), _F32),
    )(a0, a1, a2, a3, den, b, batch2d, z, wz1, bz1, wz2, bz2,
      wx0, wx1, wx2q, wx3q, wxb, bxz1, wxz2, bxz2, wxz3, bxz3)


def _edge_sc(sd3, asvec, advec, hlo, hhi, zvec, zmat, ea):
    n = asvec.shape[0]
    nbrows = sd3.shape[1]
    nb = 2 * nbrows
    k = _K
    half = n // 2
    bpt = half // _NTILES - (half // _NTILES) % 8
    last = half - bpt * (_NTILES - 1) - bpt
    dr = zvec.shape[0]
    mesh = plsc.VectorSubcoreMesh(core_axis_name="c", subcore_axis_name="s")

    @functools.partial(
        pl.kernel,
        out_type=[
            jax.ShapeDtypeStruct((4, half, 128), _F32),
            jax.ShapeDtypeStruct((_NTILES, dr, 128), _F32),
        ],
        mesh=mesh,
        compiler_params=pltpu.CompilerParams(needs_layout_passes=False),
        scratch_types=[
            pltpu.VMEM((nbrows, 128), _I32),
            pltpu.VMEM((n,), _F32),
            pltpu.VMEM((n,), _F32),
            pltpu.VMEM((dr, 128), _F32),
            [pltpu.VMEM((k,), _I32)] * 2,
            [pltpu.VMEM((k,), _I32)] * 2,
            [pltpu.VMEM((k,), _F32)] * 2,
            [pltpu.VMEM((k,), _F32)] * 2,
            [pltpu.VMEM((k, 2 * _HQ), _F32)] * 2,
            [pltpu.VMEM((k, 2 * _HQ), _F32)] * 2,
            pltpu.VMEM_SHARED((half, 128), _F32),
            [pltpu.SemaphoreType.DMA] * 2,
            [pltpu.SemaphoreType.DMA] * 2,
        ],
    )
    def ker(sd_r, as_r, ad_r, hlo_r, hhi_r, zv_r, zm_r,
            out_r, denp_r,
            sdv, asv, adv, denv, sidx, didx, eebuf, parbuf, rowb, scbuf,
            aggsh, sem1, sem2):
        c = lax.axis_index("c")
        s = lax.axis_index("s")

        pltpu.sync_copy(sd_r.at[s], sdv)
        pltpu.sync_copy(as_r, asv)
        pltpu.sync_copy(ad_r, adv)
        pltpu.sync_copy(zv_r, denv)

        zf = jnp.zeros((16,), _F32)

        @pl.loop(0, 2)
        def _phase(phase):
            off = phase * _HQ
            pltpu.sync_copy(zm_r.at[pl.ds(s * bpt, bpt)],
                            aggsh.at[pl.ds(s * bpt, bpt)])

            @pl.when(s == _NTILES - 1)
            def _():
                pltpu.sync_copy(zm_r.at[pl.ds(_NTILES * bpt, last)],
                                aggsh.at[pl.ds(_NTILES * bpt, last)])

            plsc.subcore_barrier()

            def _prep(b, i):
                for j in range(k // 16):
                    sd = sdv[b >> 1, pl.ds((b & 1) * k + j * 16, 16)]
                    sv = lax.shift_right_logical(sd, 16)
                    dv = lax.bitwise_and(sd, 0xFFFF)
                    sidx[i][pl.ds(j * 16, 16)] = sv
                    didx[i][pl.ds(j * 16, 16)] = (
                        lax.shift_right_logical(dv, 1))
                    parbuf[i][pl.ds(j * 16, 16)] = lax.convert_element_type(
                        lax.bitwise_and(dv, 1), _F32)
                    u = (plsc.load_gather(asv, [sv])
                         + plsc.load_gather(adv, [dv]))
                    e = jnp.where(u > 0, u, 0.2 * u)
                    ee = jnp.exp(e)
                    gid = (lax.broadcast((s * nb + b) * k + j * 16, (16,))
                           + lax.broadcasted_iota(_I32, (16,), 0))
                    ee = jnp.where(gid < ea, ee, 0.0)
                    eebuf[i][pl.ds(j * 16, 16)] = ee
                    plsc.addupdate_scatter(
                        denv,
                        [lax.shift_right_logical(dv, 7),
                         lax.bitwise_and(dv, 127)], ee)

            def _gather_start(i):
                @pl.when(c == 0)
                def _():
                    pltpu.async_copy(hlo_r.at[sidx[i]], rowb[i], sem1[i])

                @pl.when(c == 1)
                def _():
                    pltpu.async_copy(hhi_r.at[sidx[i]], rowb[i], sem1[i])

            def _scale_all(i):
                @pl.loop(0, k // 16)
                def _scale(j):
                    for r in range(16):
                        row = j * 16 + r
                        ridx = lax.broadcast(row, (16,))
                        spl = plsc.load_gather(eebuf[i], [ridx])
                        pv = plsc.load_gather(parbuf[i], [ridx])
                        meven = pv < 0.5
                        for u2 in range(_HQ // 16):
                            sc = rowb[i][row, pl.ds(off + u2 * 16, 16)] * spl
                            lo = jnp.where(meven, sc, zf)
                            scbuf[i][row, pl.ds(u2 * 16, 16)] = lo
                            scbuf[i][row, pl.ds(_HQ + u2 * 16, 16)] = sc - lo

            @pl.loop(0, nbrows)
            def _pair(p):
                b0 = p * 2
                _prep(b0, 0)
                _gather_start(0)
                _prep(b0 + 1, 1)
                _gather_start(1)
                pltpu.make_async_copy(hlo_r.at[sidx[0]], rowb[0],
                                      sem1[0]).wait()
                _scale_all(0)
                pltpu.async_copy(scbuf[0], aggsh.at[didx[0]], sem2[0],
                                 add=True)
                pltpu.make_async_copy(hlo_r.at[sidx[1]], rowb[1],
                                      sem1[1]).wait()
                _scale_all(1)
                pltpu.async_copy(scbuf[1], aggsh.at[didx[1]], sem2[1],
                                 add=True)
                pltpu.make_async_copy(scbuf[0], aggsh.at[didx[0]],
                                      sem2[0]).wait()
                pltpu.make_async_copy(scbuf[1], aggsh.at[didx[1]],
                                      sem2[1]).wait()

            plsc.subcore_barrier()
            q = c * 2 + phase
            pltpu.sync_copy(aggsh.at[pl.ds(s * bpt, bpt)],
                            out_r.at[q, pl.ds(s * bpt, bpt)])

            @pl.when(s == _NTILES - 1)
            def _():
                pltpu.sync_copy(aggsh.at[pl.ds(_NTILES * bpt, last)],
                                out_r.at[q, pl.ds(_NTILES * bpt, last)])

        @pl.when(c == 0)
        def _():
            pltpu.sync_copy(denv, denp_r.at[s])

    return ker(sd3, asvec, advec, hlo, hhi, zvec, zmat)


def kernel(x, z, edge_index, batch, W1, a_src1, a_dst1, b1,
           W2, a_src2, a_dst2, b2, Wz1, bz1, Wz2, bz2,
           Wxz1, bxz1, Wxz2, bxz2, Wxz3, bxz3):
    n = x.shape[0]
    hdim = W1.shape[1]
    e = edge_index.shape[1]
    ea = e + n
    chunk = _NTILES * _K
    nb = (ea + chunk - 1) // chunk
    nb = nb + (nb % 2)
    ep = nb * chunk

    loop = jnp.arange(n, dtype=edge_index.dtype)
    src = jnp.concatenate(
        [edge_index[0], loop, jnp.zeros((ep - ea,), edge_index.dtype)])
    dst = jnp.concatenate(
        [edge_index[1], loop, jnp.zeros((ep - ea,), edge_index.dtype)])
    sd3 = ((src.astype(_I32) << 16) | dst.astype(_I32)).reshape(
        _NTILES, nb // 2, 2 * _K)

    av1 = jnp.stack([a_src1, a_dst1], axis=1)
    av2 = jnp.stack([a_src2, a_dst2], axis=1)
    dr = (n + 127) // 128
    zvec = jnp.zeros((dr, 128), _F32)
    zmat = jnp.zeros((n // 2, 128), _F32)

    def _den_t(denp):
        return denp.reshape(_NTILES, dr * 128).T[:n]

    def _quarters(out4):
        return [out4[q].reshape(n, _HQ) for q in range(4)]

    hlo1, hhi1, sd1 = _feat1(x, W1, av1)
    out4_1, denp1 = _edge_sc(sd3, sd1[:, 0], sd1[:, 1],
                             hlo1, hhi1, zvec, zmat, ea)
    a1q = _quarters(out4_1)

    hlo2, hhi2, sd2 = _feat2(
        a1q[0], a1q[1], a1q[2], a1q[3], _den_t(denp1), b1[None, :],
        W2[0 * _HQ:1 * _HQ], W2[1 * _HQ:2 * _HQ],
        W2[2 * _HQ:3 * _HQ], W2[3 * _HQ:4 * _HQ], av2)
    out4_2, denp2 = _edge_sc(sd3, sd2[:, 0], sd2[:, 1],
                             hlo2, hhi2, zvec, zmat, ea)
    a2q = _quarters(out4_2)

    pred = _tail(a2q[0], a2q[1], a2q[2], a2q[3], _den_t(denp2), b2[None, :],
                 batch[None, :], z, Wz1, bz1[None, :], Wz2, bz2[None, :],
                 Wxz1[0 * _HQ:1 * _HQ], Wxz1[1 * _HQ:2 * _HQ],
                 Wxz1[2 * _HQ:3 * _HQ], Wxz1[3 * _HQ:4 * _HQ],
                 Wxz1[hdim:], bxz1[None, :],
                 Wxz2, bxz2[None, :], Wxz3, bxz3[None, :])
    return pred
